# baseline (speedup 1.0000x reference)
import functools

import jax
import jax.numpy as jnp
from jax import lax
from jax.experimental import pallas as pl
from jax.experimental.pallas import tpu as pltpu
from jax.experimental.pallas import tpu_sc as plsc

B = 16384
F = 26
V = 100000
D = 64
NC, NS, L = 2, 16, 16
NW = NC * NS
SPW = B // NW
C = 16
NCH = SPW // C

_mesh = plsc.VectorSubcoreMesh(core_axis_name="c", subcore_axis_name="s")


@functools.partial(
    pl.kernel,
    mesh=_mesh,
    out_type=jax.ShapeDtypeStruct((B, D), jnp.float32),
    scratch_types=[
        pltpu.VMEM((F * C,), jnp.int32),
        pltpu.VMEM((F, C), jnp.int32),
        pltpu.VMEM((F * C, 2 * D), jnp.float32),
        pltpu.VMEM((C, D), jnp.float32),
        pltpu.SemaphoreType.DMA,
    ],
    compiler_params=pltpu.CompilerParams(
        use_tc_tiling_on_sc=True, needs_layout_passes=False
    ),
)
def _emb(x_hbm, tab_hbm, out_hbm, idx_v, idx2_v, rows_v, out_v, sem):
    wid = lax.axis_index("s") * NC + lax.axis_index("c")

    @pl.loop(0, NCH)
    def _chunk(c):
        pltpu.sync_copy(x_hbm.at[wid, c], idx_v)
        for f in range(F):
            for k in range(C // L):
                sl = pl.ds(k * L, L)
                idx2_v[f, sl] = lax.shift_right_logical(
                    idx_v[pl.ds(f * C + k * L, L)], 1
                )
        copies = [
            pltpu.async_copy(
                tab_hbm.at[f].at[idx2_v.at[f]],
                rows_v.at[pl.ds(f * C, C), :],
                sem,
            )
            for f in range(F)
        ]
        for cp in copies:
            cp.wait()

        lanes = lax.iota(jnp.int32, L)

        @pl.loop(0, C)
        def _acc(s):
            svec = jnp.full((L,), 0, jnp.int32) + s
            accs = None
            for f in range(F):
                rsplat = svec + (f * C)
                raw = plsc.load_gather(idx_v, [rsplat])
                off = lax.shift_left((raw & 1), 6)
                vals = [
                    plsc.load_gather(rows_v, [rsplat, off + (v * L) + lanes])
                    for v in range(D // L)
                ]
                accs = vals if accs is None else [a + b for a, b in zip(accs, vals)]
            for v in range(D // L):
                out_v[s, pl.ds(v * L, L)] = accs[v]

        pltpu.sync_copy(out_v, out_hbm.at[pl.ds(wid * SPW + c * C, C), :])


def kernel(x, tables):
    xt = (
        x.astype(jnp.int32)
        .T.reshape(F, NW, NCH, C)
        .transpose(1, 2, 0, 3)
        .reshape(NW, NCH, F * C)
    )
    tab2 = tables.reshape(F, V // 2, 2 * D)
    return _emb(xt, tab2)
